# baseline (speedup 1.0000x reference)
import functools

import jax
import jax.numpy as jnp
from jax import lax
from jax.experimental import pallas as pl
from jax.experimental.pallas import tpu as pltpu
from jax.experimental.pallas import tpu_sc as plsc

S = 224
B = 4
NV = 6
N = 32768
NJOBS = B * NV
NC, NS = 2, 16
C = 2048
NG = C // 16
NCH = N // C
HC = NCH // 2
FBW = S * S
FBP = FBW + 512
DUMMY_PK = (FBW + 32) | (3 << 16) | (3 << 18)


def _splat_body(pts_ref, tbl_ref, zer_ref, out_ref,
                xa, ya, za, xb, yb, zb, pkc, ftc, fb0, fb1, cvec, tmp16,
                sema, semb, semz):
    wid = lax.axis_index("s") * NC + lax.axis_index("c")

    @pl.when(wid < NJOBS)
    def _job():
        b = wid // NV
        v = wid - b * NV

        pltpu.sync_copy(tbl_ref.at[pl.ds(v * 128, 128)], cvec)
        ca = cvec[pl.ds(0, 16)]
        sa = cvec[pl.ds(16, 16)]
        ce = cvec[pl.ds(32, 16)]
        se = cvec[pl.ds(48, 16)]
        dmin = cvec[pl.ds(64, 16)]
        dmax = cvec[pl.ds(80, 16)]

        pltpu.async_copy(zer_ref, fb0.at[pl.ds(0, FBW)], semz)
        pltpu.async_copy(zer_ref, fb1.at[pl.ds(0, FBW)], semz)

        bufs_a = (xa, ya, za)
        bufs_b = (xb, yb, zb)

        def _issue(ch, bufs, sem):
            base_in = b * 3 * N + ch * C
            for j, d in enumerate(bufs):
                pltpu.async_copy(pts_ref.at[pl.ds(base_in + j * N, C)], d, sem)

        def _wait(ch, bufs, sem):
            base_in = b * 3 * N + ch * C
            for j, d in enumerate(bufs):
                pltpu.make_async_copy(
                    pts_ref.at[pl.ds(base_in + j * N, C)], d, sem).wait()

        def _mm_chunk(bufs, mn, mx):
            x_ref, y_ref, z_ref = bufs

            def _grp(g, c2):
                mn2, mx2 = c2
                sl = pl.ds(g * 16, 16)
                x = x_ref[sl]
                y = y_ref[sl]
                z = z_ref[sl]
                z_rot = x * sa + z * ca
                zf = y * se + z_rot * ce
                return jnp.minimum(mn2, zf), jnp.maximum(mx2, zf)

            return lax.fori_loop(0, NG, _grp, (mn, mx))

        _issue(0, bufs_a, sema)
        inf = jnp.float32(jnp.inf)

        def _mm_pair(cp, carry):
            mn, mx = carry
            _issue(2 * cp + 1, bufs_b, semb)
            _wait(2 * cp, bufs_a, sema)
            mn, mx = _mm_chunk(bufs_a, mn, mx)

            @pl.when(cp < HC - 1)
            def _():
                _issue(2 * cp + 2, bufs_a, sema)

            _wait(2 * cp + 1, bufs_b, semb)
            return _mm_chunk(bufs_b, mn, mx)

        mn, mx = lax.fori_loop(
            0, HC, _mm_pair,
            (jnp.full((16,), inf, jnp.float32), jnp.full((16,), -inf, jnp.float32)))

        lane = lax.iota(jnp.int32, 16)

        def _lane_all(vec, op):
            cur = vec
            for k in (1, 2, 4, 8):
                tmp16[...] = cur
                cur = op(cur, plsc.load_gather(tmp16, [lane ^ k]))
            return cur

        zmin = _lane_all(mn, jnp.minimum)
        zmax = _lane_all(mx, jnp.maximum)
        den = zmax - zmin + 1e-6

        pltpu.make_async_copy(zer_ref, fb0.at[pl.ds(0, FBW)], semz).wait()
        pltpu.make_async_copy(zer_ref, fb1.at[pl.ds(0, FBW)], semz).wait()

        nine = lane < 9
        uvec = jnp.where(nine, lane % 3, 3)
        wvec = jnp.where(nine, lane // 3, 0)
        rvec = uvec + wvec * S

        def _splat_chunk(bufs):
            x_ref, y_ref, z_ref = bufs

            def _grp(g, cnt):
                sl = pl.ds(g * 16, 16)
                x = x_ref[sl]
                y = y_ref[sl]
                z = z_ref[sl]
                x_rot = x * ca - z * sa
                z_rot = x * sa + z * ca
                y_rot = y * ce - z_rot * se
                zf = y * se + z_rot * ce
                ft = 0.3 + 0.7 * ((zf - zmin) / den)

                def _pf(base, d):
                    return ((base + d) + 1.0) * 0.5 * (S - 1)
                fxl = _pf(x_rot, dmin)
                fxh = _pf(x_rot, dmax)
                fyl = _pf(y_rot, dmin)
                fyh = _pf(y_rot, dmax)
                keep = (fxh > -1.0) & (fxl < 224.0) & (fyh > -1.0) & (fyl < 224.0)

                def _cl(f):
                    return jnp.minimum(jnp.maximum(f, 0.0), 223.0).astype(jnp.int32)
                lo_x = _cl(fxl)
                hi_x = _cl(fxh)
                lo_y = _cl(fyl)
                hi_y = _cl(fyh)
                pk = ((lo_y * S + lo_x)
                      | ((hi_x - lo_x) << 16)
                      | ((hi_y - lo_y) << 18))
                plsc.store_compressed(pkc.at[pl.ds(cnt, 16)], pk, mask=keep)
                plsc.store_compressed(ftc.at[pl.ds(cnt, 16)], ft, mask=keep)
                inc = plsc.all_reduce_population_count(keep)
                return cnt + lax.squeeze(lax.slice(inc, (0,), (1,)), (0,))

            cnt = lax.fori_loop(0, NG, _grp, 0)
            pkc[pl.ds(cnt, 16)] = jnp.full((16,), DUMMY_PK, jnp.int32)

            def _rmw(g, _):
                base = g * 16
                pkv = pkc[pl.ds(base, 16)]
                ftv = ftc[pl.ds(base, 16)]
                for i in range(16):
                    iv = jnp.full((16,), i, jnp.int32)
                    pk = jnp.take_along_axis(pkv, iv, axis=0)
                    ft = jnp.take_along_axis(ftv, iv, axis=0)
                    ok = (uvec <= ((pk >> 16) & 3)) & (wvec <= (pk >> 18))
                    idxf = (pk & 0xFFFF) + rvec
                    f = fb0 if i % 2 == 0 else fb1
                    cur = plsc.load_gather(f, [idxf], mask=ok)
                    plsc.store_scatter(f, [idxf], jnp.maximum(cur, ft), mask=ok)
                return 0

            lax.fori_loop(0, (cnt + 15) // 16, _rmw, 0)

        _issue(0, bufs_a, sema)

        def _sp_pair(cp, _):
            _issue(2 * cp + 1, bufs_b, semb)
            _wait(2 * cp, bufs_a, sema)
            _splat_chunk(bufs_a)

            @pl.when(cp < HC - 1)
            def _():
                _issue(2 * cp + 2, bufs_a, sema)

            _wait(2 * cp + 1, bufs_b, semb)
            _splat_chunk(bufs_b)
            return 0

        lax.fori_loop(0, HC, _sp_pair, 0)

        def _mrow(r, _):
            for q in range(4):
                sl = pl.ds(r * 64 + q * 16, 16)
                fb0[sl] = jnp.maximum(fb0[sl], fb1[sl])
            return 0
        lax.fori_loop(0, FBW // 64, _mrow, 0)

        out_base = (b * NV + v) * FBW
        pltpu.sync_copy(fb0.at[pl.ds(0, FBW)], out_ref.at[pl.ds(out_base, FBW)])


@jax.jit
def kernel(points):
    az = jnp.linspace(0.0, 360.0, NV + 1)[:-1]
    el = jnp.array([0.0, 30.0, -30.0, 0.0, 0.0, 0.0])[:NV]
    azr = az * jnp.pi / 180.0
    elr = el * jnp.pi / 180.0
    offs = jnp.linspace(-2.0 / S, 2.0 / S, 5)
    dmin = jnp.full((NV,), offs[0])
    dmax = jnp.full((NV,), offs[4])
    zero = jnp.zeros((NV,))
    tbl = jnp.stack(
        [jnp.cos(azr), jnp.sin(azr), jnp.cos(elr), jnp.sin(elr),
         dmin, dmax, zero, zero], axis=1)
    tbl16 = jnp.broadcast_to(tbl[:, :, None], (NV, 8, 16))
    tbl_flat = tbl16.astype(jnp.float32).reshape(-1)
    pts_flat = points.transpose(0, 2, 1).reshape(-1)
    zbuf = jnp.zeros((FBW,), jnp.float32)

    mesh = plsc.VectorSubcoreMesh(core_axis_name="c", subcore_axis_name="s")
    run = functools.partial(
        pl.kernel,
        mesh=mesh,
        compiler_params=pltpu.CompilerParams(needs_layout_passes=False),
        out_type=jax.ShapeDtypeStruct((B * NV * FBW,), jnp.float32),
        scratch_types=[
            pltpu.VMEM((C,), jnp.float32),
            pltpu.VMEM((C,), jnp.float32),
            pltpu.VMEM((C,), jnp.float32),
            pltpu.VMEM((C,), jnp.float32),
            pltpu.VMEM((C,), jnp.float32),
            pltpu.VMEM((C,), jnp.float32),
            pltpu.VMEM((C + 16,), jnp.int32),
            pltpu.VMEM((C + 16,), jnp.float32),
            pltpu.VMEM((FBP,), jnp.float32),
            pltpu.VMEM((FBP,), jnp.float32),
            pltpu.VMEM((128,), jnp.float32),
            pltpu.VMEM((16,), jnp.float32),
            pltpu.SemaphoreType.DMA,
            pltpu.SemaphoreType.DMA,
            pltpu.SemaphoreType.DMA,
        ],
    )(_splat_body)
    img = run(pts_flat, tbl_flat, zbuf).reshape(B, NV, 1, S, S)
    return jnp.broadcast_to(img, (B, NV, 3, S, S))


# speedup vs baseline: 1.0251x; 1.0251x over previous
import functools

import jax
import jax.numpy as jnp
from jax import lax
from jax.experimental import pallas as pl
from jax.experimental.pallas import tpu as pltpu
from jax.experimental.pallas import tpu_sc as plsc

S = 224
B = 4
NV = 6
N = 32768
NJOBS = B * NV
NC, NS = 2, 16
C = 2048
NG = C // 16
NCH = N // C
HC = NCH // 2
FBW = S * S
FBP = FBW + 512
DUMMY_PK = (FBW + 32) | (3 << 16) | (3 << 18)


def _splat_body(pts_ref, tbl_ref, out_ref,
                xa, ya, za, xb, yb, zb, pkc, ftc, fb0, fb1, cvec, tmp16,
                sema, semb):
    wid = lax.axis_index("s") * NC + lax.axis_index("c")

    @pl.when(wid < NJOBS)
    def _job():
        b = wid // NV
        v = wid - b * NV

        pltpu.sync_copy(tbl_ref.at[pl.ds(v * 128, 128)], cvec)
        ca = cvec[pl.ds(0, 16)]
        sa = cvec[pl.ds(16, 16)]
        ce = cvec[pl.ds(32, 16)]
        se = cvec[pl.ds(48, 16)]
        dmin = cvec[pl.ds(64, 16)]
        dmax = cvec[pl.ds(80, 16)]

        zerov = jnp.zeros((16,), jnp.float32)

        bufs_a = (xa, ya, za)
        bufs_b = (xb, yb, zb)

        def _issue(ch, bufs, sem):
            base_in = b * 3 * N + ch * C
            for j, d in enumerate(bufs):
                pltpu.async_copy(pts_ref.at[pl.ds(base_in + j * N, C)], d, sem)

        def _wait(ch, bufs, sem):
            base_in = b * 3 * N + ch * C
            for j, d in enumerate(bufs):
                pltpu.make_async_copy(
                    pts_ref.at[pl.ds(base_in + j * N, C)], d, sem).wait()

        _issue(0, bufs_a, sema)
        _issue(1, bufs_b, semb)

        def _zrow(r, _):
            for q in range(4):
                fb0[pl.ds(r * 64 + q * 16, 16)] = zerov
                fb1[pl.ds(r * 64 + q * 16, 16)] = zerov
            return 0
        lax.fori_loop(0, FBW // 64, _zrow, 0)

        def _mm_chunk(bufs, mn, mx):
            x_ref, y_ref, z_ref = bufs

            def _grp(g, c2):
                mn2, mx2 = c2
                sl = pl.ds(g * 16, 16)
                x = x_ref[sl]
                y = y_ref[sl]
                z = z_ref[sl]
                z_rot = x * sa + z * ca
                zf = y * se + z_rot * ce
                return jnp.minimum(mn2, zf), jnp.maximum(mx2, zf)

            return lax.fori_loop(0, NG, _grp, (mn, mx))

        inf = jnp.float32(jnp.inf)

        def _mm_pair(cp, carry):
            mn, mx = carry

            @pl.when(cp > 0)
            def _():
                _issue(2 * cp + 1, bufs_b, semb)

            _wait(2 * cp, bufs_a, sema)
            mn, mx = _mm_chunk(bufs_a, mn, mx)

            @pl.when(cp < HC - 1)
            def _():
                _issue(2 * cp + 2, bufs_a, sema)

            _wait(2 * cp + 1, bufs_b, semb)
            return _mm_chunk(bufs_b, mn, mx)

        mn, mx = lax.fori_loop(
            0, HC, _mm_pair,
            (jnp.full((16,), inf, jnp.float32), jnp.full((16,), -inf, jnp.float32)))

        lane = lax.iota(jnp.int32, 16)

        def _lane_all(vec, op):
            cur = vec
            for k in (1, 2, 4, 8):
                tmp16[...] = cur
                cur = op(cur, plsc.load_gather(tmp16, [lane ^ k]))
            return cur

        zmin = _lane_all(mn, jnp.minimum)
        zmax = _lane_all(mx, jnp.maximum)
        den = zmax - zmin + 1e-6

        nine = lane < 9
        uvec = jnp.where(nine, lane % 3, 3)
        wvec = jnp.where(nine, lane // 3, 0)
        rvec = uvec + wvec * S

        def _splat_chunk(bufs):
            x_ref, y_ref, z_ref = bufs

            def _grp(g, cnt):
                sl = pl.ds(g * 16, 16)
                x = x_ref[sl]
                y = y_ref[sl]
                z = z_ref[sl]
                x_rot = x * ca - z * sa
                z_rot = x * sa + z * ca
                y_rot = y * ce - z_rot * se
                zf = y * se + z_rot * ce
                ft = 0.3 + 0.7 * ((zf - zmin) / den)

                def _pf(base, d):
                    return ((base + d) + 1.0) * 0.5 * (S - 1)
                fxl = _pf(x_rot, dmin)
                fxh = _pf(x_rot, dmax)
                fyl = _pf(y_rot, dmin)
                fyh = _pf(y_rot, dmax)
                keep = (fxh > -1.0) & (fxl < 224.0) & (fyh > -1.0) & (fyl < 224.0)

                def _cl(f):
                    return jnp.minimum(jnp.maximum(f, 0.0), 223.0).astype(jnp.int32)
                lo_x = _cl(fxl)
                hi_x = _cl(fxh)
                lo_y = _cl(fyl)
                hi_y = _cl(fyh)
                pk = ((lo_y * S + lo_x)
                      | ((hi_x - lo_x) << 16)
                      | ((hi_y - lo_y) << 18))
                plsc.store_compressed(pkc.at[pl.ds(cnt, 16)], pk, mask=keep)
                plsc.store_compressed(ftc.at[pl.ds(cnt, 16)], ft, mask=keep)
                inc = plsc.all_reduce_population_count(keep)
                return cnt + lax.squeeze(lax.slice(inc, (0,), (1,)), (0,))

            cnt = lax.fori_loop(0, NG, _grp, 0)
            pkc[pl.ds(cnt, 16)] = jnp.full((16,), DUMMY_PK, jnp.int32)

            def _rmw(g, _):
                base = g * 16
                pkv = pkc[pl.ds(base, 16)]
                ftv = ftc[pl.ds(base, 16)]
                for i in range(16):
                    iv = jnp.full((16,), i, jnp.int32)
                    pk = jnp.take_along_axis(pkv, iv, axis=0)
                    ft = jnp.take_along_axis(ftv, iv, axis=0)
                    ok = (uvec <= ((pk >> 16) & 3)) & (wvec <= (pk >> 18))
                    idxf = (pk & 0xFFFF) + rvec
                    f = fb0 if i % 2 == 0 else fb1
                    cur = plsc.load_gather(f, [idxf], mask=ok)
                    plsc.store_scatter(f, [idxf], jnp.maximum(cur, ft), mask=ok)
                return 0

            lax.fori_loop(0, (cnt + 15) // 16, _rmw, 0)

        _issue(0, bufs_a, sema)

        def _sp_pair(cp, _):
            _issue(2 * cp + 1, bufs_b, semb)
            _wait(2 * cp, bufs_a, sema)
            _splat_chunk(bufs_a)

            @pl.when(cp < HC - 1)
            def _():
                _issue(2 * cp + 2, bufs_a, sema)

            _wait(2 * cp + 1, bufs_b, semb)
            _splat_chunk(bufs_b)
            return 0

        lax.fori_loop(0, HC, _sp_pair, 0)

        def _mrow(r, _):
            for q in range(4):
                sl = pl.ds(r * 64 + q * 16, 16)
                fb0[sl] = jnp.maximum(fb0[sl], fb1[sl])
            return 0
        lax.fori_loop(0, FBW // 64, _mrow, 0)

        out_base = (b * NV + v) * FBW
        pltpu.sync_copy(fb0.at[pl.ds(0, FBW)], out_ref.at[pl.ds(out_base, FBW)])


@jax.jit
def kernel(points):
    az = jnp.linspace(0.0, 360.0, NV + 1)[:-1]
    el = jnp.array([0.0, 30.0, -30.0, 0.0, 0.0, 0.0])[:NV]
    azr = az * jnp.pi / 180.0
    elr = el * jnp.pi / 180.0
    offs = jnp.linspace(-2.0 / S, 2.0 / S, 5)
    dmin = jnp.full((NV,), offs[0])
    dmax = jnp.full((NV,), offs[4])
    zero = jnp.zeros((NV,))
    tbl = jnp.stack(
        [jnp.cos(azr), jnp.sin(azr), jnp.cos(elr), jnp.sin(elr),
         dmin, dmax, zero, zero], axis=1)
    tbl16 = jnp.broadcast_to(tbl[:, :, None], (NV, 8, 16))
    tbl_flat = tbl16.astype(jnp.float32).reshape(-1)
    pts_flat = points.transpose(0, 2, 1).reshape(-1)

    mesh = plsc.VectorSubcoreMesh(core_axis_name="c", subcore_axis_name="s")
    run = functools.partial(
        pl.kernel,
        mesh=mesh,
        compiler_params=pltpu.CompilerParams(needs_layout_passes=False),
        out_type=jax.ShapeDtypeStruct((B * NV * FBW,), jnp.float32),
        scratch_types=[
            pltpu.VMEM((C,), jnp.float32),
            pltpu.VMEM((C,), jnp.float32),
            pltpu.VMEM((C,), jnp.float32),
            pltpu.VMEM((C,), jnp.float32),
            pltpu.VMEM((C,), jnp.float32),
            pltpu.VMEM((C,), jnp.float32),
            pltpu.VMEM((C + 16,), jnp.int32),
            pltpu.VMEM((C + 16,), jnp.float32),
            pltpu.VMEM((FBP,), jnp.float32),
            pltpu.VMEM((FBP,), jnp.float32),
            pltpu.VMEM((128,), jnp.float32),
            pltpu.VMEM((16,), jnp.float32),
            pltpu.SemaphoreType.DMA,
            pltpu.SemaphoreType.DMA,
        ],
    )(_splat_body)
    img = run(pts_flat, tbl_flat).reshape(B, NV, 1, S, S)
    return jnp.broadcast_to(img, (B, NV, 3, S, S))
